# Initial kernel scaffold; baseline (speedup 1.0000x reference)
import functools

import jax
import jax.numpy as jnp
from jax import lax
from jax.experimental import pallas as pl
from jax.experimental.pallas import tpu as pltpu
from jax.experimental.pallas import tpu_sc as plsc

N = 10000
D = 128
E = 320000
NC = 2
NS = 16
NW = NC * NS
LANES = 128
CHUNKS = 80
EPT = CHUNKS * LANES
EPAD = EPT * NW
NPAD = 10016
RPT = NPAD // NS
NDUMMY = N
DEGW = 16

_mesh = plsc.VectorSubcoreMesh(core_axis_name="c", subcore_axis_name="s")


def _deg_body(dst3, zeros_w, ones_w, degp, dst_v, ones_v, acc, sem):
    cid = lax.axis_index("c")
    sid = lax.axis_index("s")
    wid = sid * NC + cid
    base = sid * RPT
    pltpu.sync_copy(zeros_w.at[pl.ds(base, RPT)], acc.at[pl.ds(base, RPT)])
    pltpu.sync_copy(dst3.at[wid], dst_v)
    pltpu.sync_copy(ones_w, ones_v)
    plsc.subcore_barrier()

    @pl.loop(0, CHUNKS, step=8)
    def _(g):
        for b in range(8):
            pltpu.async_copy(ones_v, acc.at[dst_v.at[g + b]], sem, add=True)
        for b in range(8):
            pltpu.make_async_copy(ones_v, acc.at[dst_v.at[g + b]], sem).wait()

    plsc.subcore_barrier()
    pltpu.sync_copy(acc.at[pl.ds(base, RPT)], degp.at[cid, pl.ds(base, RPT)])


_deg_call = pl.kernel(
    _deg_body,
    out_type=jax.ShapeDtypeStruct((NC, NPAD, DEGW), jnp.float32),
    mesh=_mesh,
    scratch_types=[
        pltpu.VMEM((CHUNKS, LANES), jnp.int32),
        pltpu.VMEM((LANES, DEGW), jnp.float32),
        pltpu.VMEM_SHARED((NPAD, DEGW), jnp.float32),
        pltpu.SemaphoreType.DMA,
    ],
)


def _agg_body(h, src3, dst3, zeros_d, p, src_v, dst_v, rows0, rows1, acc,
              gsem0, gsem1):
    cid = lax.axis_index("c")
    sid = lax.axis_index("s")
    wid = sid * NC + cid
    base = sid * RPT
    pltpu.sync_copy(zeros_d.at[pl.ds(base, RPT)], acc.at[pl.ds(base, RPT)])
    pltpu.sync_copy(src3.at[wid], src_v)
    pltpu.sync_copy(dst3.at[wid], dst_v)
    plsc.subcore_barrier()

    rows = (rows0, rows1)
    gsems = (gsem0, gsem1)
    pltpu.async_copy(h.at[src_v.at[0]], rows0, gsem0)

    @pl.loop(0, CHUNKS, step=2)
    def _(g):
        for b in range(2):
            j = g + b
            pltpu.make_async_copy(h.at[src_v.at[j]], rows[b], gsems[b]).wait()

            @pl.when(j + 1 < CHUNKS)
            def _():
                pltpu.async_copy(h.at[src_v.at[j + 1]], rows[1 - b],
                                 gsems[1 - b])

            pltpu.sync_copy(rows[b], acc.at[dst_v.at[j]], add=True)

    plsc.subcore_barrier()
    pltpu.sync_copy(acc.at[pl.ds(base, RPT)], p.at[cid, pl.ds(base, RPT)])


_agg_call = pl.kernel(
    _agg_body,
    out_type=jax.ShapeDtypeStruct((NC, NPAD, D), jnp.float32),
    mesh=_mesh,
    scratch_types=[
        pltpu.VMEM((CHUNKS, LANES), jnp.int32),
        pltpu.VMEM((CHUNKS, LANES), jnp.int32),
        pltpu.VMEM((LANES, D), jnp.float32),
        pltpu.VMEM((LANES, D), jnp.float32),
        pltpu.VMEM_SHARED((NPAD, D), jnp.float32),
        pltpu.SemaphoreType.DMA,
        pltpu.SemaphoreType.DMA,
    ],
)


def _tc1_body(x_ref, w1_ref, degp_ref, h1_ref, dinv_ref):
    degp = degp_ref[...]
    deg = degp[0, :, 0:1] + degp[1, :, 0:1] + 1.0
    rowid = lax.broadcasted_iota(jnp.int32, (NPAD, 1), 0)
    dinv = jnp.where(rowid < N, lax.rsqrt(deg), 0.0)
    h = jnp.dot(x_ref[...], w1_ref[...], preferred_element_type=jnp.float32)
    h1_ref[...] = h * dinv
    dinv_ref[...] = dinv


_tc1 = pl.pallas_call(
    _tc1_body,
    out_shape=(
        jax.ShapeDtypeStruct((NPAD, D), jnp.float32),
        jax.ShapeDtypeStruct((NPAD, 1), jnp.float32),
    ),
)


def _tc2_body(p_ref, h1_ref, dinv_ref, b1_ref, w2_ref, h2_ref):
    p = p_ref[...]
    s = p[0] + p[1] + h1_ref[...]
    dinv = dinv_ref[...]
    x2 = jnp.maximum(s * dinv + b1_ref[...], 0.0)
    h2 = jnp.dot(x2, w2_ref[...], preferred_element_type=jnp.float32)
    h2_ref[...] = h2 * dinv


_tc2 = pl.pallas_call(
    _tc2_body,
    out_shape=jax.ShapeDtypeStruct((NPAD, D), jnp.float32),
)


def _tc3_body(p_ref, h2_ref, dinv_ref, b2_ref, out_ref):
    p = p_ref[...]
    s = p[0] + p[1] + h2_ref[...]
    out_ref[...] = s * dinv_ref[...] + b2_ref[...]


_tc3 = pl.pallas_call(
    _tc3_body,
    out_shape=jax.ShapeDtypeStruct((NPAD, D), jnp.float32),
)


def kernel(edge_index, edge_attr, node_emb, W1, b1, W2, b2):
    src = edge_index[0].astype(jnp.int32)
    dst = edge_index[1].astype(jnp.int32)
    pad = jnp.full((EPAD - E,), NDUMMY, jnp.int32)
    src3 = jnp.concatenate([src, pad]).reshape(NW, CHUNKS, LANES)
    dst3 = jnp.concatenate([dst, pad]).reshape(NW, CHUNKS, LANES)
    xpad = jnp.zeros((NPAD, D), jnp.float32).at[:N].set(node_emb)
    zeros_d = jnp.zeros((NPAD, D), jnp.float32)
    zeros_w = jnp.zeros((NPAD, DEGW), jnp.float32)
    ones_w = jnp.ones((LANES, DEGW), jnp.float32)

    degp = _deg_call(dst3, zeros_w, ones_w)
    h1p, dinv = _tc1(xpad, W1, degp)
    p1 = _agg_call(h1p, src3, dst3, zeros_d)
    h2p = _tc2(p1, h1p, dinv, b1.reshape(1, D), W2)
    p2 = _agg_call(h2p, src3, dst3, zeros_d)
    outp = _tc3(p2, h2p, dinv, b2.reshape(1, D))
    return outp[:N]


# speedup vs baseline: 10.2546x; 10.2546x over previous
import functools

import jax
import jax.numpy as jnp
from jax import lax
from jax.experimental import pallas as pl
from jax.experimental.pallas import tpu as pltpu
from jax.experimental.pallas import tpu_sc as plsc

N = 10000
D = 128
E = 320000
NC = 2
NS = 16
NW = NC * NS
LANES = 128
CHUNKS = 80
HALF = CHUNKS // 2
EPT = CHUNKS * LANES
EPAD = EPT * NW
NPAD = 10112
RPT = NPAD // NS
NDUMMY = N
DEGW = 128

@functools.cache
def _sc_mesh():
    return plsc.VectorSubcoreMesh(core_axis_name="c", subcore_axis_name="s",
                                  num_cores=NC, num_subcores=NS)


def _deg_body(dst3, zeros_w, ones_w, degp, dst_v, ones_v, acc, sem):
    cid = lax.axis_index("c")
    sid = lax.axis_index("s")
    wid = sid * NC + cid
    base = sid * RPT
    pltpu.sync_copy(zeros_w.at[pl.ds(base, RPT)], acc.at[pl.ds(base, RPT)])
    pltpu.sync_copy(dst3.at[wid], dst_v)
    pltpu.sync_copy(ones_w, ones_v)
    plsc.subcore_barrier()

    @pl.loop(0, CHUNKS)
    def _(g):
        pltpu.sync_copy(ones_v, acc.at[dst_v.at[g]], add=True)

    plsc.subcore_barrier()
    pltpu.sync_copy(acc.at[pl.ds(base, RPT)], degp.at[cid, pl.ds(base, RPT)])


@functools.cache
def _deg_call():
    return pl.kernel(
        _deg_body,
        out_type=jax.ShapeDtypeStruct((NC, NPAD, DEGW), jnp.float32),
        mesh=_sc_mesh(),
        scratch_types=[
            pltpu.VMEM((CHUNKS, LANES), jnp.int32),
            pltpu.VMEM((LANES, DEGW), jnp.float32),
            pltpu.VMEM_SHARED((NPAD, DEGW), jnp.float32),
            pltpu.SemaphoreType.DMA,
        ],
    )


def _agg_body(h, src3, dst3, zeros_d, p, src_v, dst_v, rows0, rows1, acc,
              gsem0, gsem1):
    cid = lax.axis_index("c")
    sid = lax.axis_index("s")
    wid = sid * NC + cid
    base = sid * RPT
    pltpu.sync_copy(zeros_d.at[pl.ds(base, RPT)], acc.at[pl.ds(base, RPT)])
    pltpu.sync_copy(src3.at[wid], src_v)
    plsc.subcore_barrier()

    rows = (rows0, rows1)
    gsems = (gsem0, gsem1)
    pltpu.async_copy(h.at[src_v.at[0]], rows0, gsem0)

    for half in range(2):
        pltpu.sync_copy(dst3.at[wid, pl.ds(half * HALF, HALF)], dst_v)

        @pl.loop(0, HALF, step=2)
        def _(g):
            for b in range(2):
                jl = g + b
                j = half * HALF + jl
                pltpu.make_async_copy(h.at[src_v.at[j]], rows[b],
                                      gsems[b]).wait()

                @pl.when(j + 1 < CHUNKS)
                def _():
                    pltpu.async_copy(h.at[src_v.at[j + 1]], rows[1 - b],
                                     gsems[1 - b])

                pltpu.sync_copy(rows[b], acc.at[dst_v.at[jl]], add=True)

    plsc.subcore_barrier()
    pltpu.sync_copy(acc.at[pl.ds(base, RPT)], p.at[cid, pl.ds(base, RPT)])


@functools.cache
def _agg_call():
    return pl.kernel(
        _agg_body,
        out_type=jax.ShapeDtypeStruct((NC, NPAD, D), jnp.float32),
        mesh=_sc_mesh(),
        scratch_types=[
            pltpu.VMEM((CHUNKS, LANES), jnp.int32),
            pltpu.VMEM((HALF, LANES), jnp.int32),
            pltpu.VMEM((LANES, D), jnp.float32),
            pltpu.VMEM((LANES, D), jnp.float32),
            pltpu.VMEM_SHARED((NPAD, D), jnp.float32),
            pltpu.SemaphoreType.DMA,
            pltpu.SemaphoreType.DMA,
        ],
    )


def _tc1_body(x_ref, w1_ref, degp_ref, h1_ref, dinv_ref):
    degp = degp_ref[...]
    deg = degp[0, :, 0:1] + degp[1, :, 0:1] + 1.0
    rowid = lax.broadcasted_iota(jnp.int32, (NPAD, 1), 0)
    dinv = jnp.where(rowid < N, lax.rsqrt(deg), 0.0)
    h = jnp.dot(x_ref[...], w1_ref[...], preferred_element_type=jnp.float32)
    h1_ref[...] = h * dinv
    dinv_ref[...] = dinv


_tc1 = pl.pallas_call(
    _tc1_body,
    out_shape=(
        jax.ShapeDtypeStruct((NPAD, D), jnp.float32),
        jax.ShapeDtypeStruct((NPAD, 1), jnp.float32),
    ),
)


def _tc2_body(p_ref, h1_ref, dinv_ref, b1_ref, w2_ref, h2_ref):
    p = p_ref[...]
    s = p[0] + p[1] + h1_ref[...]
    dinv = dinv_ref[...]
    x2 = jnp.maximum(s * dinv + b1_ref[...], 0.0)
    h2 = jnp.dot(x2, w2_ref[...], preferred_element_type=jnp.float32)
    h2_ref[...] = h2 * dinv


_tc2 = pl.pallas_call(
    _tc2_body,
    out_shape=jax.ShapeDtypeStruct((NPAD, D), jnp.float32),
)


def _tc3_body(p_ref, h2_ref, dinv_ref, b2_ref, out_ref):
    p = p_ref[...]
    s = p[0] + p[1] + h2_ref[...]
    out_ref[...] = s * dinv_ref[...] + b2_ref[...]


_tc3 = pl.pallas_call(
    _tc3_body,
    out_shape=jax.ShapeDtypeStruct((NPAD, D), jnp.float32),
)


def kernel(edge_index, edge_attr, node_emb, W1, b1, W2, b2):
    src = edge_index[0].astype(jnp.int32)
    dst = edge_index[1].astype(jnp.int32)
    pad = jnp.full((EPAD - E,), NDUMMY, jnp.int32)
    src3 = jnp.concatenate([src, pad]).reshape(NW, CHUNKS, LANES)
    dst3 = jnp.concatenate([dst, pad]).reshape(NW, CHUNKS, LANES)
    xpad = jnp.zeros((NPAD, D), jnp.float32).at[:N].set(node_emb)
    zeros_d = jnp.zeros((NPAD, D), jnp.float32)
    zeros_w = jnp.zeros((NPAD, DEGW), jnp.float32)
    ones_w = jnp.ones((LANES, DEGW), jnp.float32)

    degp = _deg_call()(dst3, zeros_w, ones_w)
    h1p, dinv = _tc1(xpad, W1, degp)
    p1 = _agg_call()(h1p, src3, dst3, zeros_d)
    h2p = _tc2(p1, h1p, dinv, b1.reshape(1, D), W2)
    p2 = _agg_call()(h2p, src3, dst3, zeros_d)
    outp = _tc3(p2, h2p, dinv, b2.reshape(1, D))
    return outp[:N]
